# baseline (speedup 1.0000x reference)
import jax
import jax.numpy as jnp
from jax import lax
from jax.experimental import pallas as pl
from jax.experimental.pallas import tpu as pltpu
from jax.experimental.pallas import tpu_sc as plsc

N = 10000
E = 320000
D = 128
HD = D // 2

NC = 2
NS = 16
EPT = E // NS
CHUNK = 80
NCHUNK = EPT // CHUNK
NBUF = 5
SB = 50
SROUND = SB // NBUF
NSB = NCHUNK // SB
RPS = 624
RPS_LAST = N - (NS - 1) * RPS

_mesh = plsc.VectorSubcoreMesh(
    core_axis_name="c", subcore_axis_name="s", num_cores=NC, num_subcores=NS
)


def _agg_body(xs_hbm, src_hbm, dst_hbm, zero_hbm, out_hbm,
              src_v, dst_v, rows_v, gsem, ssem, acc_sh, x_sh):
    c = lax.axis_index("c")
    s = lax.axis_index("s")

    @pl.when(s < NS - 1)
    def _():
        pltpu.sync_copy(zero_hbm.at[pl.ds(s * RPS, RPS)],
                        acc_sh.at[pl.ds(s * RPS, RPS)])
        pltpu.sync_copy(xs_hbm.at[c, pl.ds(s * RPS, RPS)],
                        x_sh.at[pl.ds(s * RPS, RPS)])

    @pl.when(s == NS - 1)
    def _():
        pltpu.sync_copy(zero_hbm.at[pl.ds((NS - 1) * RPS, RPS_LAST)],
                        acc_sh.at[pl.ds((NS - 1) * RPS, RPS_LAST)])
        pltpu.sync_copy(xs_hbm.at[c, pl.ds((NS - 1) * RPS, RPS_LAST)],
                        x_sh.at[pl.ds((NS - 1) * RPS, RPS_LAST)])

    pltpu.sync_copy(src_hbm.at[s, pl.ds(0, SB)], src_v)
    pltpu.sync_copy(dst_hbm.at[s, pl.ds(0, SB)], dst_v)
    plsc.subcore_barrier()

    xc = x_sh

    def sb_body(p, carry):
        for b in range(NBUF):
            pltpu.async_copy(xc.at[src_v.at[b]], rows_v.at[b], gsem.at[b])

        def round_body(g, cc):
            for b in range(NBUF):
                lj = g * NBUF + b
                pltpu.make_async_copy(xc.at[src_v.at[lj]], rows_v.at[b],
                                      gsem.at[b]).wait()
                pltpu.async_copy(rows_v.at[b], acc_sh.at[dst_v.at[lj]],
                                 ssem.at[b], add=True)
            for b in range(NBUF):
                ljn = (g + 1) * NBUF + b

                @pl.when(ljn < SB)
                def _():
                    pltpu.make_async_copy(rows_v.at[b],
                                          acc_sh.at[dst_v.at[ljn]],
                                          ssem.at[b]).wait()
                    pltpu.async_copy(xc.at[src_v.at[ljn]], rows_v.at[b],
                                     gsem.at[b])
            return cc

        lax.fori_loop(0, SROUND, round_body, 0)
        for b in range(NBUF):
            pltpu.make_async_copy(rows_v.at[b], acc_sh.at[dst_v.at[b]],
                                  ssem.at[b]).wait()

        @pl.when(p + 1 < NSB)
        def _():
            pltpu.sync_copy(src_hbm.at[s, pl.ds((p + 1) * SB, SB)], src_v)
            pltpu.sync_copy(dst_hbm.at[s, pl.ds((p + 1) * SB, SB)], dst_v)

        return carry

    lax.fori_loop(0, NSB, sb_body, 0)
    plsc.subcore_barrier()

    @pl.when(s < NS - 1)
    def _():
        pltpu.sync_copy(acc_sh.at[pl.ds(s * RPS, RPS)],
                        out_hbm.at[c, pl.ds(s * RPS, RPS)])

    @pl.when(s == NS - 1)
    def _():
        pltpu.sync_copy(acc_sh.at[pl.ds((NS - 1) * RPS, RPS_LAST)],
                        out_hbm.at[c, pl.ds((NS - 1) * RPS, RPS_LAST)])


_agg = pl.kernel(
    _agg_body,
    out_type=jax.ShapeDtypeStruct((NC, N, HD), jnp.float32),
    mesh=_mesh,
    scratch_types=[
        pltpu.VMEM((SB, CHUNK), jnp.int32),
        pltpu.VMEM((SB, CHUNK), jnp.int32),
        pltpu.VMEM((NBUF, CHUNK, HD), jnp.float32),
        pltpu.SemaphoreType.DMA((NBUF,)),
        pltpu.SemaphoreType.DMA((NBUF,)),
        pltpu.VMEM_SHARED((N, HD), jnp.float32),
        pltpu.VMEM_SHARED((N, HD), jnp.float32),
    ],
    compiler_params=pltpu.CompilerParams(use_tc_tiling_on_sc=False),
)

R = 400
GRID = N // R


def _mlp1_body(x_ref, p_ref, w1_ref, b1_ref, w2_ref, b2_ref, o_ref):
    agg = jnp.concatenate([p_ref[0], p_ref[1]], axis=-1)
    sgm = x_ref[...] + agg
    t = jnp.dot(sgm, w1_ref[...], preferred_element_type=jnp.float32)
    t = jnp.maximum(t + b1_ref[...], 0.0)
    h = jnp.dot(t, w2_ref[...], preferred_element_type=jnp.float32)
    h = jnp.maximum(h + b2_ref[...], 0.0)
    o_ref[0] = h[:, :HD]
    o_ref[1] = h[:, HD:]


_mlp1 = pl.pallas_call(
    _mlp1_body,
    grid=(GRID,),
    in_specs=[
        pl.BlockSpec((R, D), lambda i: (i, 0)),
        pl.BlockSpec((NC, R, HD), lambda i: (0, i, 0)),
        pl.BlockSpec((D, D), lambda i: (0, 0)),
        pl.BlockSpec((1, D), lambda i: (0, 0)),
        pl.BlockSpec((D, D), lambda i: (0, 0)),
        pl.BlockSpec((1, D), lambda i: (0, 0)),
    ],
    out_specs=pl.BlockSpec((NC, R, HD), lambda i: (0, i, 0)),
    out_shape=jax.ShapeDtypeStruct((NC, N, HD), jnp.float32),
)


def _mlp2_body(h_ref, p_ref, w1_ref, b1_ref, o_ref):
    i = pl.program_id(0)
    h = jnp.concatenate([h_ref[0], h_ref[1]], axis=-1)
    agg = jnp.concatenate([p_ref[0], p_ref[1]], axis=-1)
    sgm = h + agg
    g = jnp.dot(sgm, w1_ref[...], preferred_element_type=jnp.float32)
    g = jnp.maximum(g + b1_ref[...], 0.0)
    part = jnp.sum(g, axis=0, keepdims=True)

    @pl.when(i == 0)
    def _():
        o_ref[...] = jnp.zeros_like(o_ref)

    o_ref[...] += part


_mlp2 = pl.pallas_call(
    _mlp2_body,
    grid=(GRID,),
    in_specs=[
        pl.BlockSpec((NC, R, HD), lambda i: (0, i, 0)),
        pl.BlockSpec((NC, R, HD), lambda i: (0, i, 0)),
        pl.BlockSpec((D, D), lambda i: (0, 0)),
        pl.BlockSpec((1, D), lambda i: (0, 0)),
    ],
    out_specs=pl.BlockSpec((1, D), lambda i: (0, 0)),
    out_shape=jax.ShapeDtypeStruct((1, D), jnp.float32),
)


def _head_body(cs_ref, w2_ref, b2_ref, o_ref):
    v = cs_ref[...] * (1.0 / N)
    o_ref[...] = jnp.dot(v, w2_ref[...],
                         preferred_element_type=jnp.float32) + b2_ref[...]


_head = pl.pallas_call(
    _head_body,
    out_shape=jax.ShapeDtypeStruct((1, D), jnp.float32),
)


def kernel(x, edge_index, W1a, b1a, W2a, b2a, W1b, b1b, W2b, b2b, batch_size):
    src_r = edge_index[0].reshape(NS, NCHUNK, CHUNK)
    dst_r = edge_index[1].reshape(NS, NCHUNK, CHUNK)
    zeros = jnp.zeros((N, HD), jnp.float32)
    b1a_, b2a_, b1b_, b2b_ = (b.reshape(1, D) for b in (b1a, b2a, b1b, b2b))
    xs = jnp.stack([x[:, :HD], x[:, HD:]], axis=0)

    p1 = _agg(xs, src_r, dst_r, zeros)
    hs = _mlp1(x, p1, W1a, b1a_, W2a, b2a_)
    p2 = _agg(hs, src_r, dst_r, zeros)
    cs = _mlp2(hs, p2, W1b, b1b_)
    out = _head(cs, W2b, b2b_)
    return out.reshape(-1)


# speedup vs baseline: 1.4376x; 1.4376x over previous
import jax
import jax.numpy as jnp
from jax import lax
from jax.experimental import pallas as pl
from jax.experimental.pallas import tpu as pltpu
from jax.experimental.pallas import tpu_sc as plsc

N = 10000
E = 320000
D = 128

NC = 2
NS = 16
NW = NC * NS
EPW = E // NW
CHUNK = 40
NCHUNK = EPW // CHUNK
NBUF = 5
SB = 50
SROUND = SB // NBUF
NSB = NCHUNK // SB
RPS = 624
RPS_LAST = N - (NS - 1) * RPS

_mesh = plsc.VectorSubcoreMesh(
    core_axis_name="c", subcore_axis_name="s", num_cores=NC, num_subcores=NS
)


def _agg_body(x_hbm, src_hbm, dst_hbm, zero_hbm, out_hbm,
              src_v, dst_v, rows_v, gsem, ssem, acc_sh):
    c = lax.axis_index("c")
    s = lax.axis_index("s")
    w = c * NS + s

    @pl.when(s < NS - 1)
    def _():
        pltpu.sync_copy(zero_hbm.at[pl.ds(s * RPS, RPS)],
                        acc_sh.at[pl.ds(s * RPS, RPS)])

    @pl.when(s == NS - 1)
    def _():
        pltpu.sync_copy(zero_hbm.at[pl.ds((NS - 1) * RPS, RPS_LAST)],
                        acc_sh.at[pl.ds((NS - 1) * RPS, RPS_LAST)])

    pltpu.sync_copy(src_hbm.at[w, 0], src_v)
    pltpu.sync_copy(dst_hbm.at[w, 0], dst_v)
    plsc.subcore_barrier()

    def sb_body(p, carry):
        for b in range(NBUF):
            pltpu.async_copy(x_hbm.at[src_v.at[b]], rows_v.at[b], gsem.at[b])

        def round_body(g, cc):
            for b in range(NBUF):
                lj = g * NBUF + b
                pltpu.make_async_copy(x_hbm.at[src_v.at[lj]], rows_v.at[b],
                                      gsem.at[b]).wait()
                pltpu.async_copy(rows_v.at[b], acc_sh.at[dst_v.at[lj]],
                                 ssem.at[b], add=True)
            for b in range(NBUF):
                ljn = (g + 1) * NBUF + b

                @pl.when(ljn < SB)
                def _():
                    pltpu.make_async_copy(rows_v.at[b],
                                          acc_sh.at[dst_v.at[ljn]],
                                          ssem.at[b]).wait()
                    pltpu.async_copy(x_hbm.at[src_v.at[ljn]], rows_v.at[b],
                                     gsem.at[b])
            return cc

        lax.fori_loop(0, SROUND, round_body, 0)
        for b in range(NBUF):
            pltpu.make_async_copy(rows_v.at[b], acc_sh.at[dst_v.at[b]],
                                  ssem.at[b]).wait()

        @pl.when(p + 1 < NSB)
        def _():
            pltpu.sync_copy(src_hbm.at[w, p + 1], src_v)
            pltpu.sync_copy(dst_hbm.at[w, p + 1], dst_v)

        return carry

    lax.fori_loop(0, NSB, sb_body, 0)
    plsc.subcore_barrier()

    @pl.when(s < NS - 1)
    def _():
        pltpu.sync_copy(acc_sh.at[pl.ds(s * RPS, RPS)],
                        out_hbm.at[c, pl.ds(s * RPS, RPS)])

    @pl.when(s == NS - 1)
    def _():
        pltpu.sync_copy(acc_sh.at[pl.ds((NS - 1) * RPS, RPS_LAST)],
                        out_hbm.at[c, pl.ds((NS - 1) * RPS, RPS_LAST)])


_agg = pl.kernel(
    _agg_body,
    out_type=jax.ShapeDtypeStruct((NC, N, D), jnp.float32),
    mesh=_mesh,
    scratch_types=[
        pltpu.VMEM((SB, CHUNK), jnp.int32),
        pltpu.VMEM((SB, CHUNK), jnp.int32),
        pltpu.VMEM((NBUF, CHUNK, D), jnp.float32),
        pltpu.SemaphoreType.DMA((NBUF,)),
        pltpu.SemaphoreType.DMA((NBUF,)),
        pltpu.VMEM_SHARED((N, D), jnp.float32),
    ],
)

R = 400
GRID = N // R


def _mlp1_body(x_ref, p_ref, w1_ref, b1_ref, w2_ref, b2_ref, o_ref):
    sgm = x_ref[...] + p_ref[0] + p_ref[1]
    t = jnp.dot(sgm, w1_ref[...], preferred_element_type=jnp.float32)
    t = jnp.maximum(t + b1_ref[...], 0.0)
    h = jnp.dot(t, w2_ref[...], preferred_element_type=jnp.float32)
    o_ref[...] = jnp.maximum(h + b2_ref[...], 0.0)


_mlp1 = pl.pallas_call(
    _mlp1_body,
    grid=(GRID,),
    in_specs=[
        pl.BlockSpec((R, D), lambda i: (i, 0)),
        pl.BlockSpec((NC, R, D), lambda i: (0, i, 0)),
        pl.BlockSpec((D, D), lambda i: (0, 0)),
        pl.BlockSpec((1, D), lambda i: (0, 0)),
        pl.BlockSpec((D, D), lambda i: (0, 0)),
        pl.BlockSpec((1, D), lambda i: (0, 0)),
    ],
    out_specs=pl.BlockSpec((R, D), lambda i: (i, 0)),
    out_shape=jax.ShapeDtypeStruct((N, D), jnp.float32),
)


def _mlp2_body(h_ref, p_ref, w1_ref, b1_ref, o_ref):
    i = pl.program_id(0)
    sgm = h_ref[...] + p_ref[0] + p_ref[1]
    g = jnp.dot(sgm, w1_ref[...], preferred_element_type=jnp.float32)
    g = jnp.maximum(g + b1_ref[...], 0.0)
    part = jnp.sum(g, axis=0, keepdims=True)

    @pl.when(i == 0)
    def _():
        o_ref[...] = jnp.zeros_like(o_ref)

    o_ref[...] += part


_mlp2 = pl.pallas_call(
    _mlp2_body,
    grid=(GRID,),
    in_specs=[
        pl.BlockSpec((R, D), lambda i: (i, 0)),
        pl.BlockSpec((NC, R, D), lambda i: (0, i, 0)),
        pl.BlockSpec((D, D), lambda i: (0, 0)),
        pl.BlockSpec((1, D), lambda i: (0, 0)),
    ],
    out_specs=pl.BlockSpec((1, D), lambda i: (0, 0)),
    out_shape=jax.ShapeDtypeStruct((1, D), jnp.float32),
)


def _head_body(cs_ref, w2_ref, b2_ref, o_ref):
    v = cs_ref[...] * (1.0 / N)
    o_ref[...] = jnp.dot(v, w2_ref[...],
                         preferred_element_type=jnp.float32) + b2_ref[...]


_head = pl.pallas_call(
    _head_body,
    out_shape=jax.ShapeDtypeStruct((1, D), jnp.float32),
)


def kernel(x, edge_index, W1a, b1a, W2a, b2a, W1b, b1b, W2b, b2b, batch_size):
    src_r = edge_index[0].reshape(NW, NSB, SB, CHUNK)
    dst_r = edge_index[1].reshape(NW, NSB, SB, CHUNK)
    zeros = jnp.zeros((N, D), jnp.float32)
    b1a_, b2a_, b1b_, b2b_ = (b.reshape(1, D) for b in (b1a, b2a, b1b, b2b))

    p1 = _agg(x, src_r, dst_r, zeros)
    h = _mlp1(x, p1, W1a, b1a_, W2a, b2a_)
    p2 = _agg(h, src_r, dst_r, zeros)
    cs = _mlp2(h, p2, W1b, b1b_)
    out = _head(cs, W2b, b2b_)
    return out.reshape(-1)


# speedup vs baseline: 1.4438x; 1.0043x over previous
import jax
import jax.numpy as jnp
from jax import lax
from jax.experimental import pallas as pl
from jax.experimental.pallas import tpu as pltpu
from jax.experimental.pallas import tpu_sc as plsc

N = 10000
E = 320000
D = 128

NC = 2
NS = 16
NW = NC * NS
EPW = E // NW
CHUNK = 40
NCHUNK = EPW // CHUNK
NBUF = 5
SB = 50
SROUND = SB // NBUF
NSB = NCHUNK // SB
RPS = 624
RPS_LAST = N - (NS - 1) * RPS

_mesh = plsc.VectorSubcoreMesh(
    core_axis_name="c", subcore_axis_name="s", num_cores=NC, num_subcores=NS
)


def _agg_body(x_hbm, src_hbm, dst_hbm, zero_hbm, out_hbm,
              src_v, dst_v, rows_v, gsem, ssem, acc_sh):
    c = lax.axis_index("c")
    s = lax.axis_index("s")
    w = c * NS + s

    @pl.when(s < NS - 1)
    def _():
        pltpu.sync_copy(zero_hbm.at[pl.ds(s * RPS, RPS)],
                        acc_sh.at[pl.ds(s * RPS, RPS)])

    @pl.when(s == NS - 1)
    def _():
        pltpu.sync_copy(zero_hbm.at[pl.ds((NS - 1) * RPS, RPS_LAST)],
                        acc_sh.at[pl.ds((NS - 1) * RPS, RPS_LAST)])

    pltpu.sync_copy(src_hbm.at[w, 0], src_v)
    pltpu.sync_copy(dst_hbm.at[w, 0], dst_v)
    plsc.subcore_barrier()

    def sb_body(p, carry):
        for b in range(NBUF):
            pltpu.async_copy(x_hbm.at[src_v.at[b]], rows_v.at[b], gsem.at[b])

        def round_body(g, cc):
            for b in range(NBUF):
                lj = g * NBUF + b
                pltpu.make_async_copy(x_hbm.at[src_v.at[lj]], rows_v.at[b],
                                      gsem.at[b]).wait()
                pltpu.async_copy(rows_v.at[b], acc_sh.at[dst_v.at[lj]],
                                 ssem.at[b], add=True)
            for b in range(NBUF):
                ljn = (g + 1) * NBUF + b

                @pl.when(ljn < SB)
                def _():
                    pltpu.make_async_copy(rows_v.at[b],
                                          acc_sh.at[dst_v.at[ljn]],
                                          ssem.at[b]).wait()
                    pltpu.async_copy(x_hbm.at[src_v.at[ljn]], rows_v.at[b],
                                     gsem.at[b])
            return cc

        lax.fori_loop(0, SROUND, round_body, 0)
        for b in range(NBUF):
            pltpu.make_async_copy(rows_v.at[b], acc_sh.at[dst_v.at[b]],
                                  ssem.at[b]).wait()

        @pl.when(p + 1 < NSB)
        def _():
            pltpu.sync_copy(src_hbm.at[w, p + 1], src_v)
            pltpu.sync_copy(dst_hbm.at[w, p + 1], dst_v)

        return carry

    lax.fori_loop(0, NSB, sb_body, 0)
    plsc.subcore_barrier()

    @pl.when(s < NS - 1)
    def _():
        pltpu.sync_copy(acc_sh.at[pl.ds(s * RPS, RPS)],
                        out_hbm.at[c, pl.ds(s * RPS, RPS)])

    @pl.when(s == NS - 1)
    def _():
        pltpu.sync_copy(acc_sh.at[pl.ds((NS - 1) * RPS, RPS_LAST)],
                        out_hbm.at[c, pl.ds((NS - 1) * RPS, RPS_LAST)])


_agg = pl.kernel(
    _agg_body,
    out_type=jax.ShapeDtypeStruct((NC, N, D), jnp.float32),
    mesh=_mesh,
    scratch_types=[
        pltpu.VMEM((SB, CHUNK), jnp.int32),
        pltpu.VMEM((SB, CHUNK), jnp.int32),
        pltpu.VMEM((NBUF, CHUNK, D), jnp.float32),
        pltpu.SemaphoreType.DMA((NBUF,)),
        pltpu.SemaphoreType.DMA((NBUF,)),
        pltpu.VMEM_SHARED((N, D), jnp.float32),
    ],
)

R = 400
GRID = N // R


def _mlp1_body(x_ref, p_ref, w1_ref, b1_ref, w2_ref, b2_ref, o_ref):
    sgm = x_ref[...] + p_ref[0] + p_ref[1]
    t = jnp.dot(sgm, w1_ref[...], preferred_element_type=jnp.float32)
    t = jnp.maximum(t + b1_ref[...], 0.0)
    h = jnp.dot(t, w2_ref[...], preferred_element_type=jnp.float32)
    o_ref[...] = jnp.maximum(h + b2_ref[...], 0.0)


_mlp1 = pl.pallas_call(
    _mlp1_body,
    grid=(GRID,),
    in_specs=[
        pl.BlockSpec((R, D), lambda i: (i, 0)),
        pl.BlockSpec((NC, R, D), lambda i: (0, i, 0)),
        pl.BlockSpec((D, D), lambda i: (0, 0)),
        pl.BlockSpec((1, D), lambda i: (0, 0)),
        pl.BlockSpec((D, D), lambda i: (0, 0)),
        pl.BlockSpec((1, D), lambda i: (0, 0)),
    ],
    out_specs=pl.BlockSpec((R, D), lambda i: (i, 0)),
    out_shape=jax.ShapeDtypeStruct((N, D), jnp.float32),
)


def _mlp2_body(h_ref, p_ref, w1_ref, b1_ref, w2_ref, b2_ref, cs_ref, o_ref):
    i = pl.program_id(0)
    sgm = h_ref[...] + p_ref[0] + p_ref[1]
    g = jnp.dot(sgm, w1_ref[...], preferred_element_type=jnp.float32)
    g = jnp.maximum(g + b1_ref[...], 0.0)
    part = jnp.sum(g, axis=0, keepdims=True)

    @pl.when(i == 0)
    def _():
        cs_ref[...] = jnp.zeros_like(cs_ref)

    cs_ref[...] += part

    @pl.when(i == GRID - 1)
    def _():
        v = cs_ref[...] * (1.0 / N)
        o_ref[...] = jnp.dot(v, w2_ref[...],
                             preferred_element_type=jnp.float32) + b2_ref[...]


_mlp2 = pl.pallas_call(
    _mlp2_body,
    grid=(GRID,),
    in_specs=[
        pl.BlockSpec((R, D), lambda i: (i, 0)),
        pl.BlockSpec((NC, R, D), lambda i: (0, i, 0)),
        pl.BlockSpec((D, D), lambda i: (0, 0)),
        pl.BlockSpec((1, D), lambda i: (0, 0)),
        pl.BlockSpec((D, D), lambda i: (0, 0)),
        pl.BlockSpec((1, D), lambda i: (0, 0)),
    ],
    out_specs=[
        pl.BlockSpec((1, D), lambda i: (0, 0)),
        pl.BlockSpec((1, D), lambda i: (0, 0)),
    ],
    out_shape=[
        jax.ShapeDtypeStruct((1, D), jnp.float32),
        jax.ShapeDtypeStruct((1, D), jnp.float32),
    ],
)


def kernel(x, edge_index, W1a, b1a, W2a, b2a, W1b, b1b, W2b, b2b, batch_size):
    src_r = edge_index[0].reshape(NW, NSB, SB, CHUNK)
    dst_r = edge_index[1].reshape(NW, NSB, SB, CHUNK)
    zeros = jnp.zeros((N, D), jnp.float32)
    b1a_, b2a_, b1b_, b2b_ = (b.reshape(1, D) for b in (b1a, b2a, b1b, b2b))

    p1 = _agg(x, src_r, dst_r, zeros)
    h = _mlp1(x, p1, W1a, b1a_, W2a, b2a_)
    p2 = _agg(h, src_r, dst_r, zeros)
    _, out = _mlp2(h, p2, W1b, b1b_, W2b, b2b_)
    return out.reshape(-1)


# speedup vs baseline: 1.4688x; 1.0173x over previous
import jax
import jax.numpy as jnp
from jax import lax
from jax.experimental import pallas as pl
from jax.experimental.pallas import tpu as pltpu
from jax.experimental.pallas import tpu_sc as plsc

N = 10000
E = 320000
D = 128

NC = 2
NS = 16
NW = NC * NS
EPW = E // NW
CHUNK = 80
NCHUNK = EPW // CHUNK
NBUF = 4
SB = 25
SROUND = -(-SB // NBUF)
NSB = NCHUNK // SB
RPS = 624
RPS_LAST = N - (NS - 1) * RPS

_mesh = plsc.VectorSubcoreMesh(
    core_axis_name="c", subcore_axis_name="s", num_cores=NC, num_subcores=NS
)


def _agg_body(x_hbm, src_hbm, dst_hbm, zero_hbm, out_hbm,
              src_v, dst_v, rows_v, gsem, ssem, acc_sh):
    c = lax.axis_index("c")
    s = lax.axis_index("s")
    w = c * NS + s

    @pl.when(s < NS - 1)
    def _():
        pltpu.sync_copy(zero_hbm.at[pl.ds(s * RPS, RPS)],
                        acc_sh.at[pl.ds(s * RPS, RPS)])

    @pl.when(s == NS - 1)
    def _():
        pltpu.sync_copy(zero_hbm.at[pl.ds((NS - 1) * RPS, RPS_LAST)],
                        acc_sh.at[pl.ds((NS - 1) * RPS, RPS_LAST)])

    pltpu.sync_copy(src_hbm.at[w, 0], src_v)
    pltpu.sync_copy(dst_hbm.at[w, 0], dst_v)
    plsc.subcore_barrier()

    def sb_body(p, carry):
        for b in range(NBUF):
            pltpu.async_copy(x_hbm.at[src_v.at[b]], rows_v.at[b], gsem.at[b])

        def round_body(g, cc):
            for b in range(NBUF):
                lj = g * NBUF + b

                @pl.when(lj < SB)
                def _():
                    pltpu.make_async_copy(x_hbm.at[src_v.at[lj]],
                                          rows_v.at[b], gsem.at[b]).wait()
                    pltpu.async_copy(rows_v.at[b], acc_sh.at[dst_v.at[lj]],
                                     ssem.at[b], add=True)
            for b in range(NBUF):
                ljn = (g + 1) * NBUF + b

                @pl.when(ljn < SB)
                def _():
                    pltpu.make_async_copy(rows_v.at[b],
                                          acc_sh.at[dst_v.at[ljn]],
                                          ssem.at[b]).wait()
                    pltpu.async_copy(x_hbm.at[src_v.at[ljn]], rows_v.at[b],
                                     gsem.at[b])
            return cc

        lax.fori_loop(0, SROUND, round_body, 0)
        for b in range(NBUF):
            pltpu.make_async_copy(rows_v.at[b], acc_sh.at[dst_v.at[b]],
                                  ssem.at[b]).wait()

        @pl.when(p + 1 < NSB)
        def _():
            pltpu.sync_copy(src_hbm.at[w, p + 1], src_v)
            pltpu.sync_copy(dst_hbm.at[w, p + 1], dst_v)

        return carry

    lax.fori_loop(0, NSB, sb_body, 0)
    plsc.subcore_barrier()

    @pl.when(s < NS - 1)
    def _():
        pltpu.sync_copy(acc_sh.at[pl.ds(s * RPS, RPS)],
                        out_hbm.at[c, pl.ds(s * RPS, RPS)])

    @pl.when(s == NS - 1)
    def _():
        pltpu.sync_copy(acc_sh.at[pl.ds((NS - 1) * RPS, RPS_LAST)],
                        out_hbm.at[c, pl.ds((NS - 1) * RPS, RPS_LAST)])


_agg = pl.kernel(
    _agg_body,
    out_type=jax.ShapeDtypeStruct((NC, N, D), jnp.float32),
    mesh=_mesh,
    scratch_types=[
        pltpu.VMEM((SB, CHUNK), jnp.int32),
        pltpu.VMEM((SB, CHUNK), jnp.int32),
        pltpu.VMEM((NBUF, CHUNK, D), jnp.float32),
        pltpu.SemaphoreType.DMA((NBUF,)),
        pltpu.SemaphoreType.DMA((NBUF,)),
        pltpu.VMEM_SHARED((N, D), jnp.float32),
    ],
)

R = 400
GRID = N // R


def _mlp1_body(x_ref, p_ref, w1_ref, b1_ref, w2_ref, b2_ref, o_ref):
    sgm = x_ref[...] + p_ref[0] + p_ref[1]
    t = jnp.dot(sgm, w1_ref[...], preferred_element_type=jnp.float32)
    t = jnp.maximum(t + b1_ref[...], 0.0)
    h = jnp.dot(t, w2_ref[...], preferred_element_type=jnp.float32)
    o_ref[...] = jnp.maximum(h + b2_ref[...], 0.0)


_mlp1 = pl.pallas_call(
    _mlp1_body,
    grid=(GRID,),
    in_specs=[
        pl.BlockSpec((R, D), lambda i: (i, 0)),
        pl.BlockSpec((NC, R, D), lambda i: (0, i, 0)),
        pl.BlockSpec((D, D), lambda i: (0, 0)),
        pl.BlockSpec((1, D), lambda i: (0, 0)),
        pl.BlockSpec((D, D), lambda i: (0, 0)),
        pl.BlockSpec((1, D), lambda i: (0, 0)),
    ],
    out_specs=pl.BlockSpec((R, D), lambda i: (i, 0)),
    out_shape=jax.ShapeDtypeStruct((N, D), jnp.float32),
)


def _mlp2_body(h_ref, p_ref, w1_ref, b1_ref, w2_ref, b2_ref, cs_ref, o_ref):
    i = pl.program_id(0)
    sgm = h_ref[...] + p_ref[0] + p_ref[1]
    g = jnp.dot(sgm, w1_ref[...], preferred_element_type=jnp.float32)
    g = jnp.maximum(g + b1_ref[...], 0.0)
    part = jnp.sum(g, axis=0, keepdims=True)

    @pl.when(i == 0)
    def _():
        cs_ref[...] = jnp.zeros_like(cs_ref)

    cs_ref[...] += part

    @pl.when(i == GRID - 1)
    def _():
        v = cs_ref[...] * (1.0 / N)
        o_ref[...] = jnp.dot(v, w2_ref[...],
                             preferred_element_type=jnp.float32) + b2_ref[...]


_mlp2 = pl.pallas_call(
    _mlp2_body,
    grid=(GRID,),
    in_specs=[
        pl.BlockSpec((R, D), lambda i: (i, 0)),
        pl.BlockSpec((NC, R, D), lambda i: (0, i, 0)),
        pl.BlockSpec((D, D), lambda i: (0, 0)),
        pl.BlockSpec((1, D), lambda i: (0, 0)),
        pl.BlockSpec((D, D), lambda i: (0, 0)),
        pl.BlockSpec((1, D), lambda i: (0, 0)),
    ],
    out_specs=[
        pl.BlockSpec((1, D), lambda i: (0, 0)),
        pl.BlockSpec((1, D), lambda i: (0, 0)),
    ],
    out_shape=[
        jax.ShapeDtypeStruct((1, D), jnp.float32),
        jax.ShapeDtypeStruct((1, D), jnp.float32),
    ],
)


def kernel(x, edge_index, W1a, b1a, W2a, b2a, W1b, b1b, W2b, b2b, batch_size):
    src_r = edge_index[0].reshape(NW, NSB, SB, CHUNK)
    dst_r = edge_index[1].reshape(NW, NSB, SB, CHUNK)
    zeros = jnp.zeros((N, D), jnp.float32)
    b1a_, b2a_, b1b_, b2b_ = (b.reshape(1, D) for b in (b1a, b2a, b1b, b2b))

    p1 = _agg(x, src_r, dst_r, zeros)
    h = _mlp1(x, p1, W1a, b1a_, W2a, b2a_)
    p2 = _agg(h, src_r, dst_r, zeros)
    _, out = _mlp2(h, p2, W1b, b1b_, W2b, b2b_)
    return out.reshape(-1)


# speedup vs baseline: 1.4861x; 1.0118x over previous
import jax
import jax.numpy as jnp
from jax import lax
from jax.experimental import pallas as pl
from jax.experimental.pallas import tpu as pltpu
from jax.experimental.pallas import tpu_sc as plsc

N = 10000
E = 320000
D = 128

NC = 2
NS = 16
NW = NC * NS
EPW = E // NW
CHUNK = 80
NCHUNK = EPW // CHUNK
NBUF = 4
SB = 25
SROUND = -(-SB // NBUF)
NSB = NCHUNK // SB
RPS = 624
RPS_LAST = N - (NS - 1) * RPS

_mesh = plsc.VectorSubcoreMesh(
    core_axis_name="c", subcore_axis_name="s", num_cores=NC, num_subcores=NS
)


def _agg_body(x_hbm, src_hbm, dst_hbm, zero_hbm, out_hbm,
              src_v, dst_v, rows_v, gsem, ssem, acc_sh):
    c = lax.axis_index("c")
    s = lax.axis_index("s")
    w = c * NS + s

    @pl.when(s < NS - 1)
    def _():
        pltpu.sync_copy(zero_hbm.at[pl.ds(s * RPS, RPS)],
                        acc_sh.at[pl.ds(s * RPS, RPS)])

    @pl.when(s == NS - 1)
    def _():
        pltpu.sync_copy(zero_hbm.at[pl.ds((NS - 1) * RPS, RPS_LAST)],
                        acc_sh.at[pl.ds((NS - 1) * RPS, RPS_LAST)])

    pltpu.sync_copy(src_hbm.at[w, 0], src_v)
    pltpu.sync_copy(dst_hbm.at[w, 0], dst_v)
    plsc.subcore_barrier()

    def sb_body(p, carry):
        for b in range(NBUF):
            pltpu.async_copy(x_hbm.at[src_v.at[b]], rows_v.at[b], gsem.at[b])

        def round_body(g, cc):
            for b in range(NBUF):
                lj = g * NBUF + b

                @pl.when(lj < SB)
                def _():
                    pltpu.make_async_copy(x_hbm.at[src_v.at[lj]],
                                          rows_v.at[b], gsem.at[b]).wait()
                    pltpu.async_copy(rows_v.at[b], acc_sh.at[dst_v.at[lj]],
                                     ssem.at[b], add=True)
            for b in range(NBUF):
                ljn = (g + 1) * NBUF + b

                @pl.when(ljn < SB)
                def _():
                    pltpu.make_async_copy(rows_v.at[b],
                                          acc_sh.at[dst_v.at[ljn]],
                                          ssem.at[b]).wait()
                    pltpu.async_copy(x_hbm.at[src_v.at[ljn]], rows_v.at[b],
                                     gsem.at[b])
            return cc

        lax.fori_loop(0, SROUND, round_body, 0)
        for b in range(NBUF):
            pltpu.make_async_copy(rows_v.at[b], acc_sh.at[dst_v.at[b]],
                                  ssem.at[b]).wait()

        @pl.when(p + 1 < NSB)
        def _():
            pltpu.sync_copy(src_hbm.at[w, p + 1], src_v)
            pltpu.sync_copy(dst_hbm.at[w, p + 1], dst_v)

        return carry

    lax.fori_loop(0, NSB, sb_body, 0)
    plsc.subcore_barrier()

    @pl.when(s < NS - 1)
    def _():
        pltpu.sync_copy(acc_sh.at[pl.ds(s * RPS, RPS)],
                        out_hbm.at[c, pl.ds(s * RPS, RPS)])

    @pl.when(s == NS - 1)
    def _():
        pltpu.sync_copy(acc_sh.at[pl.ds((NS - 1) * RPS, RPS_LAST)],
                        out_hbm.at[c, pl.ds((NS - 1) * RPS, RPS_LAST)])


_agg = pl.kernel(
    _agg_body,
    out_type=jax.ShapeDtypeStruct((NC, N, D), jnp.float32),
    mesh=_mesh,
    scratch_types=[
        pltpu.VMEM((SB, CHUNK), jnp.int32),
        pltpu.VMEM((SB, CHUNK), jnp.int32),
        pltpu.VMEM((NBUF, CHUNK, D), jnp.float32),
        pltpu.SemaphoreType.DMA((NBUF,)),
        pltpu.SemaphoreType.DMA((NBUF,)),
        pltpu.VMEM_SHARED((N, D), jnp.float32),
    ],
    compiler_params=pltpu.CompilerParams(use_tc_tiling_on_sc=False),
)

R = 400
GRID = N // R


def _mlp1_body(x_ref, p_ref, w1_ref, b1_ref, w2_ref, b2_ref, o_ref):
    sgm = x_ref[...] + p_ref[0] + p_ref[1]
    t = jnp.dot(sgm, w1_ref[...], preferred_element_type=jnp.float32)
    t = jnp.maximum(t + b1_ref[...], 0.0)
    h = jnp.dot(t, w2_ref[...], preferred_element_type=jnp.float32)
    o_ref[...] = jnp.maximum(h + b2_ref[...], 0.0)


_mlp1 = pl.pallas_call(
    _mlp1_body,
    grid=(GRID,),
    in_specs=[
        pl.BlockSpec((R, D), lambda i: (i, 0)),
        pl.BlockSpec((NC, R, D), lambda i: (0, i, 0)),
        pl.BlockSpec((D, D), lambda i: (0, 0)),
        pl.BlockSpec((1, D), lambda i: (0, 0)),
        pl.BlockSpec((D, D), lambda i: (0, 0)),
        pl.BlockSpec((1, D), lambda i: (0, 0)),
    ],
    out_specs=pl.BlockSpec((R, D), lambda i: (i, 0)),
    out_shape=jax.ShapeDtypeStruct((N, D), jnp.float32),
)


def _mlp2_body(h_ref, p_ref, w1_ref, b1_ref, w2_ref, b2_ref, cs_ref, o_ref):
    i = pl.program_id(0)
    sgm = h_ref[...] + p_ref[0] + p_ref[1]
    g = jnp.dot(sgm, w1_ref[...], preferred_element_type=jnp.float32)
    g = jnp.maximum(g + b1_ref[...], 0.0)
    part = jnp.sum(g, axis=0, keepdims=True)

    @pl.when(i == 0)
    def _():
        cs_ref[...] = jnp.zeros_like(cs_ref)

    cs_ref[...] += part

    @pl.when(i == GRID - 1)
    def _():
        v = cs_ref[...] * (1.0 / N)
        o_ref[...] = jnp.dot(v, w2_ref[...],
                             preferred_element_type=jnp.float32) + b2_ref[...]


_mlp2 = pl.pallas_call(
    _mlp2_body,
    grid=(GRID,),
    in_specs=[
        pl.BlockSpec((R, D), lambda i: (i, 0)),
        pl.BlockSpec((NC, R, D), lambda i: (0, i, 0)),
        pl.BlockSpec((D, D), lambda i: (0, 0)),
        pl.BlockSpec((1, D), lambda i: (0, 0)),
        pl.BlockSpec((D, D), lambda i: (0, 0)),
        pl.BlockSpec((1, D), lambda i: (0, 0)),
    ],
    out_specs=[
        pl.BlockSpec((1, D), lambda i: (0, 0)),
        pl.BlockSpec((1, D), lambda i: (0, 0)),
    ],
    out_shape=[
        jax.ShapeDtypeStruct((1, D), jnp.float32),
        jax.ShapeDtypeStruct((1, D), jnp.float32),
    ],
)


def kernel(x, edge_index, W1a, b1a, W2a, b2a, W1b, b1b, W2b, b2b, batch_size):
    src_r = edge_index[0].reshape(NW, NSB, SB, CHUNK)
    dst_r = edge_index[1].reshape(NW, NSB, SB, CHUNK)
    zeros = jnp.zeros((N, D), jnp.float32)
    b1a_, b2a_, b1b_, b2b_ = (b.reshape(1, D) for b in (b1a, b2a, b1b, b2b))

    p1 = _agg(x, src_r, dst_r, zeros)
    h = _mlp1(x, p1, W1a, b1a_, W2a, b2a_)
    p2 = _agg(h, src_r, dst_r, zeros)
    _, out = _mlp2(h, p2, W1b, b1b_, W2b, b2b_)
    return out.reshape(-1)
